# baseline (speedup 1.0000x reference)
import jax
import jax.numpy as jnp
from jax import lax
from jax.experimental import pallas as pl
from jax.experimental.pallas import tpu as pltpu
from jax.experimental.pallas import tpu_sc as plsc

_CW = 512
_G = _CW // 16
_BKB = 2048


def _sc_body(xt_hbm, lab_hbm, th_hbm, w_hbm, xt_v, lab_v, w_v, th_v, in_sem, out_sem):
    c = lax.axis_index("c")
    s = lax.axis_index("s")
    wid = s * 2 + c
    c0 = wid * _CW
    in_copies = [
        pltpu.async_copy(xt_hbm.at[:, pl.ds(c0, _CW)], xt_v, in_sem),
        pltpu.async_copy(lab_hbm.at[pl.ds(c0, _CW)], lab_v, in_sem),
        pltpu.async_copy(th_hbm, th_v, in_sem),
    ]
    for cp in in_copies:
        cp.wait()
    cond = th_v[...] < 0.5
    one = jnp.full((16,), 1.0, jnp.float32)
    zero = jnp.zeros((16,), jnp.float32)
    for k in range(_G):
        sl = pl.ds(k * 16, 16)
        a = xt_v[0, sl]
        b = xt_v[1, sl]
        cc = xt_v[2, sl]
        d = xt_v[3, sl]
        e = xt_v[4, sl]
        lo = jnp.maximum(jnp.minimum(a, b), jnp.minimum(cc, d))
        hi = jnp.minimum(jnp.maximum(a, b), jnp.maximum(cc, d))
        med = jnp.maximum(jnp.minimum(lo, hi), jnp.minimum(jnp.maximum(lo, hi), e))
        lab = lab_v[sl]
        rows = (a, b, cc, d, e)
        for j in range(5):
            xj = rows[j]
            branch_a = jnp.where(xj >= med, xj, zero)
            branch_b = jnp.where(lab == j, one, zero)
            w_v[j, sl] = jnp.where(cond, branch_a, branch_b)
    pltpu.async_copy(w_v, w_hbm.at[:, pl.ds(c0, _CW)], out_sem).wait()


def _fill_body(o_ref):
    o_ref[...] = jnp.full(o_ref.shape, -1.0, jnp.float32)


def _splice_body(w_ref, filled_ref, o_ref):
    del filled_ref
    w = w_ref[...]
    o_ref[...] = jnp.full(o_ref.shape, -1.0, jnp.float32)
    o_ref[:, 0:1, :] = w.reshape(5, 1, w.shape[-1])


def kernel(inputs_0, inputs_1, inputs_2, inputs_3, inputs_4):
    n = inputs_0.shape[0]
    xt = inputs_0.T
    mesh = plsc.VectorSubcoreMesh(core_axis_name="c", subcore_axis_name="s")
    w = pl.kernel(
        _sc_body,
        out_type=jax.ShapeDtypeStruct((5, n), jnp.float32),
        mesh=mesh,
        scratch_types=[
            pltpu.VMEM((5, _CW), jnp.float32),
            pltpu.VMEM((_CW,), jnp.int32),
            pltpu.VMEM((5, _CW), jnp.float32),
            pltpu.VMEM((16,), jnp.float32),
            pltpu.SemaphoreType.DMA,
            pltpu.SemaphoreType.DMA,
        ],
        compiler_params=pltpu.CompilerParams(use_tc_tiling_on_sc=True),
    )(xt, inputs_1, jnp.broadcast_to(inputs_4, (16,)))
    filled = pl.pallas_call(
        _fill_body,
        grid=(n // _BKB,),
        in_specs=[],
        out_specs=pl.BlockSpec((5, 100, _BKB), lambda i: (0, 0, i)),
        out_shape=jax.ShapeDtypeStruct((5, 100, n), jnp.float32),
    )()
    out_t = pl.pallas_call(
        _splice_body,
        grid=(n // _BKB,),
        in_specs=[
            pl.BlockSpec((5, _BKB), lambda i: (0, i)),
            pl.BlockSpec((5, 8, _BKB), lambda i: (0, 0, i)),
        ],
        out_specs=pl.BlockSpec((5, 8, _BKB), lambda i: (0, 0, i)),
        out_shape=jax.ShapeDtypeStruct((5, 100, n), jnp.float32),
        input_output_aliases={1: 0},
    )(w, filled)
    return jnp.transpose(out_t, (2, 1, 0))


# speedup vs baseline: 1.1120x; 1.1120x over previous
import jax
import jax.numpy as jnp
from jax import lax
from jax.experimental import pallas as pl
from jax.experimental.pallas import tpu as pltpu
from jax.experimental.pallas import tpu_sc as plsc

_CW = 512
_G = _CW // 16


def _sc_body(xt_hbm, lab_hbm, th_hbm, out_hbm, xt_v, lab_v, w_v, stage_v, th_v,
             in_sem, out_sem):
    c = lax.axis_index("c")
    s = lax.axis_index("s")
    wid = s * 2 + c
    c0 = wid * _CW
    in_copies = [
        pltpu.async_copy(xt_hbm.at[:, pl.ds(c0, _CW)], xt_v, in_sem),
        pltpu.async_copy(lab_hbm.at[pl.ds(c0, _CW)], lab_v, in_sem),
        pltpu.async_copy(th_hbm, th_v, in_sem),
    ]
    neg = jnp.full((16,), -1.0, jnp.float32)

    def fill_row(r, _):
        for k in range(_CW // 16):
            stage_v[r, pl.ds(k * 16, 16)] = neg
        return 0

    lax.fori_loop(0, 92, fill_row, 0)

    out_copies = []
    for j in range(5):
        out_copies.append(
            pltpu.async_copy(stage_v, out_hbm.at[j, pl.ds(8, 92), pl.ds(c0, _CW)],
                             out_sem)
        )

    def fill_head(j, _):
        def fill_head_row(r, _):
            for k in range(_CW // 16):
                w_v[j, r, pl.ds(k * 16, 16)] = neg
            return 0

        lax.fori_loop(1, 8, fill_head_row, 0)
        return 0

    lax.fori_loop(0, 5, fill_head, 0)

    for cp in in_copies:
        cp.wait()
    cond = th_v[...] < 0.5

    for k in range(_G):
        sl = pl.ds(k * 16, 16)
        a = xt_v[0, sl]
        b = xt_v[1, sl]
        cc = xt_v[2, sl]
        d = xt_v[3, sl]
        e = xt_v[4, sl]
        lo = jnp.maximum(jnp.minimum(a, b), jnp.minimum(cc, d))
        hi = jnp.minimum(jnp.maximum(a, b), jnp.maximum(cc, d))
        med = jnp.maximum(jnp.minimum(lo, hi), jnp.minimum(jnp.maximum(lo, hi), e))
        lab = lab_v[sl]
        one = jnp.full((16,), 1.0, jnp.float32)
        zero = jnp.zeros((16,), jnp.float32)
        rows = (a, b, cc, d, e)
        for j in range(5):
            xj = rows[j]
            branch_a = jnp.where(xj >= med, xj, zero)
            branch_b = jnp.where(lab == j, one, zero)
            w_v[j, 0, sl] = jnp.where(cond, branch_a, branch_b)

    for j in range(5):
        out_copies.append(
            pltpu.async_copy(w_v.at[j], out_hbm.at[j, pl.ds(0, 8), pl.ds(c0, _CW)],
                             out_sem)
        )
    for cp in out_copies:
        cp.wait()


def kernel(inputs_0, inputs_1, inputs_2, inputs_3, inputs_4):
    n = inputs_0.shape[0]
    xt = inputs_0.T
    mesh = plsc.VectorSubcoreMesh(core_axis_name="c", subcore_axis_name="s")
    out_t = pl.kernel(
        _sc_body,
        out_type=jax.ShapeDtypeStruct((5, 100, n), jnp.float32),
        mesh=mesh,
        scratch_types=[
            pltpu.VMEM((5, _CW), jnp.float32),
            pltpu.VMEM((_CW,), jnp.int32),
            pltpu.VMEM((5, 8, _CW), jnp.float32),
            pltpu.VMEM((92, _CW), jnp.float32),
            pltpu.VMEM((16,), jnp.float32),
            pltpu.SemaphoreType.DMA,
            pltpu.SemaphoreType.DMA,
        ],
        compiler_params=pltpu.CompilerParams(use_tc_tiling_on_sc=True),
    )(xt, inputs_1, jnp.broadcast_to(inputs_4, (16,)))
    return jnp.transpose(out_t, (2, 1, 0))
